# Initial kernel scaffold; baseline (speedup 1.0000x reference)
import jax
import jax.numpy as jnp
from jax.experimental import pallas as pl


def kernel(x, W1, a1, W2, a2):
    raise NotImplementedError("write your pallas kernel here")


# speedup vs baseline: 2.0348x; 2.0348x over previous
import functools

import jax
import jax.numpy as jnp
from jax import lax
from jax.experimental import pallas as pl
from jax.experimental.pallas import tpu as pltpu

ALPHA = 0.2
N = 2048
F = 16
ROW_BLK = 256


def _fused_gat_body(x_ref, w1_ref, a1_ref, w2_ref, a2_ref, o_ref, wh_s, h_s):
    xb = x_ref[0]

    def layer(xin, W, a, write_out):
        wh = jnp.dot(xin, W, preferred_element_type=jnp.float32)
        wh_s[...] = wh
        f1 = jnp.dot(wh, a[:F, :], preferred_element_type=jnp.float32)
        f2r = lax.dot_general(
            a[F:, :], wh,
            dimension_numbers=(((0,), (1,)), ((), ())),
            preferred_element_type=jnp.float32,
        )
        for j in range(N // ROW_BLK):
            f1b = f1[j * ROW_BLK:(j + 1) * ROW_BLK, :]
            e = f1b + f2r
            e = jnp.where(e >= 0, e, ALPHA * e)
            m = jnp.max(e, axis=1, keepdims=True)
            p = jnp.exp(e - m)
            s = jnp.sum(p, axis=1, keepdims=True)
            h = jnp.dot(p, wh_s[...], preferred_element_type=jnp.float32) / s
            write_out(j, jnp.where(h > 0, h, jnp.exp(h) - 1.0))

    def write_h(j, val):
        h_s[pl.ds(j * ROW_BLK, ROW_BLK), :] = val

    def write_o(j, val):
        o_ref[0, pl.ds(j * ROW_BLK, ROW_BLK), :] = val

    layer(xb, w1_ref[...], a1_ref[...], write_h)
    layer(h_s[...], w2_ref[...], a2_ref[...], write_o)


@jax.jit
def kernel(x, W1, a1, W2, a2):
    B = x.shape[0]
    grid = (B,)
    return pl.pallas_call(
        _fused_gat_body,
        grid=grid,
        in_specs=[
            pl.BlockSpec((1, N, F), lambda b: (b, 0, 0)),
            pl.BlockSpec((F, F), lambda b: (0, 0)),
            pl.BlockSpec((2 * F, 1), lambda b: (0, 0)),
            pl.BlockSpec((F, F), lambda b: (0, 0)),
            pl.BlockSpec((2 * F, 1), lambda b: (0, 0)),
        ],
        out_specs=pl.BlockSpec((1, N, F), lambda b: (b, 0, 0)),
        out_shape=jax.ShapeDtypeStruct((B, N, F), jnp.float32),
        scratch_shapes=[
            pltpu.VMEM((N, F), jnp.float32),
            pltpu.VMEM((N, F), jnp.float32),
        ],
    )(x, W1, a1, W2, a2)


# speedup vs baseline: 2.4598x; 1.2088x over previous
import functools

import jax
import jax.numpy as jnp
from jax import lax
from jax.experimental import pallas as pl
from jax.experimental.pallas import tpu as pltpu

ALPHA = 0.2
N = 2048
F = 16
ROW_BLK = 256


def _fused_gat_body(x_ref, w1_ref, a1_ref, w2_ref, a2_ref, o_ref, wh_s, h_s):
    xb = x_ref[0]

    def layer(xin, W, a, write_out):
        wh = jnp.dot(xin, W, preferred_element_type=jnp.float32)
        wh_s[...] = wh
        f1 = jnp.dot(wh, a[:F, :], preferred_element_type=jnp.float32)
        f2c = jnp.dot(wh, a[F:, :], preferred_element_type=jnp.float32)
        f2r = lax.dot_general(
            a[F:, :], wh,
            dimension_numbers=(((0,), (1,)), ((), ())),
            preferred_element_type=jnp.float32,
        )
        u = jnp.exp(f1)
        v = jnp.exp(ALPHA * f1)
        g = jnp.exp(f2c)
        gh = jnp.exp(ALPHA * f2c)
        ones = jnp.ones((N, 1), jnp.float32)
        qp = jnp.concatenate([wh, ones], axis=1) * g
        qn = jnp.concatenate([wh, ones], axis=1) * gh
        tn = jnp.sum(qn, axis=0, keepdims=True)
        for j in range(N // ROW_BLK):
            sl = slice(j * ROW_BLK, (j + 1) * ROW_BLK)
            e = f1[sl, :] + f2r
            mask = jnp.where(e >= 0, 1.0, 0.0)
            mp = jnp.dot(mask, qp, preferred_element_type=jnp.float32)
            mn = jnp.dot(mask, qn, preferred_element_type=jnp.float32)
            mn = tn - mn
            num = u[sl, :] * mp[:, :F] + v[sl, :] * mn[:, :F]
            den = u[sl, :] * mp[:, F:] + v[sl, :] * mn[:, F:]
            h = num / den
            write_out(j, jnp.where(h > 0, h, jnp.exp(h) - 1.0))

    def write_h(j, val):
        h_s[pl.ds(j * ROW_BLK, ROW_BLK), :] = val

    def write_o(j, val):
        o_ref[0, pl.ds(j * ROW_BLK, ROW_BLK), :] = val

    layer(xb, w1_ref[...], a1_ref[...], write_h)
    layer(h_s[...], w2_ref[...], a2_ref[...], write_o)


@jax.jit
def kernel(x, W1, a1, W2, a2):
    B = x.shape[0]
    grid = (B,)
    return pl.pallas_call(
        _fused_gat_body,
        grid=grid,
        in_specs=[
            pl.BlockSpec((1, N, F), lambda b: (b, 0, 0)),
            pl.BlockSpec((F, F), lambda b: (0, 0)),
            pl.BlockSpec((2 * F, 1), lambda b: (0, 0)),
            pl.BlockSpec((F, F), lambda b: (0, 0)),
            pl.BlockSpec((2 * F, 1), lambda b: (0, 0)),
        ],
        out_specs=pl.BlockSpec((1, N, F), lambda b: (b, 0, 0)),
        out_shape=jax.ShapeDtypeStruct((B, N, F), jnp.float32),
        scratch_shapes=[
            pltpu.VMEM((N, F), jnp.float32),
            pltpu.VMEM((N, F), jnp.float32),
        ],
    )(x, W1, a1, W2, a2)
